# baseline (speedup 1.0000x reference)
import jax
import jax.numpy as jnp
from jax import lax
from jax.experimental import pallas as pl
from jax.experimental.pallas import tpu as pltpu
from jax.experimental.pallas import tpu_sc as plsc

WORD_VOCAB = 1000000
CHAR_VOCAB = 262
EMB = 64
B = 4096
L_SEQ = 50
C = 16

NC, NS, LANES = 2, 16, 16
NW = NC * NS
N = B * L_SEQ
PER_W = N // NW
CH = 128
NCHUNK = PER_W // CH
NG = CH // LANES
EBP = EMB // 2
REP = 4
PSTR = CH + 1


def _body(wid, cids2, glove, ctab, out,
          idx0, idx1, cid0, cid1, rows0, rows1, comb0, comb1, pooled_v, ct_v,
          s_idx0, s_idx1, s_cid0, s_cid1, s_g0, s_g1, s_r0, s_r1, s_o0, s_o1):
    idx_v = (idx0, idx1)
    cid_v = (cid0, cid1)
    rows_v = (rows0, rows1)
    comb_v = (comb0, comb1)
    s_idx = (s_idx0, s_idx1)
    s_cid = (s_cid0, s_cid1)
    s_g = (s_g0, s_g1)
    s_r = (s_r0, s_r1)
    s_o = (s_o0, s_o1)

    w = lax.axis_index("s") * NC + lax.axis_index("c")
    pltpu.sync_copy(ctab, ct_v)
    ebase = [lax.iota(jnp.int32, LANES) * PSTR + j * LANES * PSTR
             for j in range(EMB // LANES)]
    lm = jnp.bitwise_and(lax.iota(jnp.int32, LANES), REP - 1)

    def fire_in(kc, p):
        p0 = pl.multiple_of(w * PER_W + kc * CH, CH)
        pltpu.async_copy(wid.at[pl.ds(p0, CH)], idx_v[p], s_idx[p])
        l = lax.shift_right_logical(p0, 12)
        b0 = pl.multiple_of(jnp.bitwise_and(p0, B - 1), CH)
        pltpu.async_copy(cids2.at[pl.ds(l * C, C), pl.ds(b0, CH)],
                         cid_v[p], s_cid[p])

    def wait_in(p):
        pltpu.make_async_copy(wid.at[pl.ds(0, CH)], idx_v[p], s_idx[p]).wait()
        pltpu.make_async_copy(cids2.at[pl.ds(0, C), pl.ds(0, CH)],
                              cid_v[p], s_cid[p]).wait()

    def fire_glove(p):
        pltpu.async_copy(glove.at[idx_v[p]], rows_v[p], s_g[p])

    def wait_glove(p):
        pltpu.make_async_copy(glove.at[idx_v[p]], rows_v[p], s_g[p]).wait()

    def fire_rows_out(kc, p):
        p0 = w * PER_W + kc * CH
        pltpu.async_copy(rows_v[p], out.at[pl.ds(p0, CH)], s_r[p])

    def wait_rows_out(p):
        pltpu.make_async_copy(rows_v[p], out.at[pl.ds(0, CH)], s_r[p]).wait()

    def fire_out(kc, p):
        p0 = w * PER_W + kc * CH
        pltpu.async_copy(comb_v[p], out.at[pl.ds(p0, CH), pl.ds(EMB, EMB)],
                         s_o[p])

    def wait_out(p):
        pltpu.make_async_copy(comb_v[p], out.at[pl.ds(0, CH), pl.ds(EMB, EMB)],
                              s_o[p]).wait()

    def compute(p):
        cv = cid_v[p]
        ov = comb_v[p]

        def group_body(g, gc):
            gw = g * LANES
            for ebb in range(EBP // 8):
                accs = [None] * 8
                for c in range(C):
                    ids = cv[c, pl.ds(gw, LANES)]
                    base_c = ids * REP + lm
                    for q in range(8):
                        eb = ebb * 8 + q
                        v = plsc.bitcast(
                            plsc.load_gather(
                                ct_v, [base_c + eb * (CHAR_VOCAB * REP)]),
                            jnp.bfloat16)
                        accs[q] = v if c == 0 else jnp.maximum(accs[q], v)
                for q in range(8):
                    eb = ebb * 8 + q
                    ai = plsc.bitcast(accs[q], jnp.int32)
                    lo = plsc.bitcast(lax.shift_left(ai, 16), jnp.float32)
                    hi = plsc.bitcast(jnp.bitwise_and(ai, jnp.int32(-65536)),
                                      jnp.float32)
                    pooled_v[pl.ds((2 * eb) * PSTR + gw, LANES)] = lo
                    pooled_v[pl.ds((2 * eb + 1) * PSTR + gw, LANES)] = hi
            for i in range(LANES):
                word = gw + i
                for j in range(EMB // LANES):
                    v = plsc.load_gather(pooled_v, [ebase[j] + word])
                    ov[word, pl.ds(j * LANES, LANES)] = v
            return gc

        lax.fori_loop(0, NG, group_body, 0)

    fire_in(0, 0)
    wait_in(0)
    fire_glove(0)
    fire_in(1, 1)

    def super_body(kk, carry):
        for ph in range(2):
            k = kk * 2 + ph
            p, p1 = ph, 1 - ph
            if ph == 0:
                wait_in(p1)

                @pl.when(kk >= 1)
                def _():
                    wait_out(p1)

                fire_glove(p1)
            else:
                @pl.when(kk < (NCHUNK // 2) - 1)
                def _():
                    wait_in(p1)
                    wait_out(p1)
                    fire_glove(p1)

            wait_glove(p)
            fire_rows_out(k, p)
            compute(p)
            wait_rows_out(p)
            fire_out(k, p)

            @pl.when(kk < (NCHUNK // 2) - 1)
            def _():
                fire_in(k + 2, p)
        return carry

    lax.fori_loop(0, NCHUNK // 2, super_body, 0)
    wait_out(0)
    wait_out(1)


def kernel(word_ids, char_ids, glove_table, char_table):
    wid = word_ids.T.reshape(N)
    glove128 = jnp.pad(glove_table, ((0, 0), (0, EMB)))
    cids2 = char_ids.transpose(1, 2, 0).reshape(L_SEQ * C, B)
    ct_bf = char_table.astype(jnp.bfloat16).reshape(CHAR_VOCAB, EBP, 2)
    ct_pk = lax.bitcast_convert_type(ct_bf, jnp.int32)
    ct_pk = jnp.repeat(ct_pk.T.reshape(EBP * CHAR_VOCAB), REP)

    mesh = plsc.VectorSubcoreMesh(core_axis_name="c", subcore_axis_name="s")
    out = pl.kernel(
        _body,
        out_type=jax.ShapeDtypeStruct((N, 2 * EMB), jnp.float32),
        mesh=mesh,
        compiler_params=pltpu.CompilerParams(
            needs_layout_passes=False, use_tc_tiling_on_sc=False),
        scratch_types=[
            pltpu.VMEM((CH,), jnp.int32),
            pltpu.VMEM((CH,), jnp.int32),
            pltpu.VMEM((C, CH), jnp.int32),
            pltpu.VMEM((C, CH), jnp.int32),
            pltpu.VMEM((CH, 2 * EMB), jnp.float32),
            pltpu.VMEM((CH, 2 * EMB), jnp.float32),
            pltpu.VMEM((CH, EMB), jnp.float32),
            pltpu.VMEM((CH, EMB), jnp.float32),
            pltpu.VMEM((EMB * PSTR,), jnp.float32),
            pltpu.VMEM((CHAR_VOCAB * EBP * REP,), jnp.int32),
            pltpu.SemaphoreType.DMA,
            pltpu.SemaphoreType.DMA,
            pltpu.SemaphoreType.DMA,
            pltpu.SemaphoreType.DMA,
            pltpu.SemaphoreType.DMA,
            pltpu.SemaphoreType.DMA,
            pltpu.SemaphoreType.DMA,
            pltpu.SemaphoreType.DMA,
            pltpu.SemaphoreType.DMA,
            pltpu.SemaphoreType.DMA,
        ],
    )(wid, cids2, glove128, ct_pk)
    return out.reshape(L_SEQ, B, 2 * EMB).transpose(1, 0, 2)


# speedup vs baseline: 1.0626x; 1.0626x over previous
import jax
import jax.numpy as jnp
from jax import lax
from jax.experimental import pallas as pl
from jax.experimental.pallas import tpu as pltpu
from jax.experimental.pallas import tpu_sc as plsc

WORD_VOCAB = 1000000
CHAR_VOCAB = 262
EMB = 64
B = 4096
L_SEQ = 50
C = 16

NC, NS, LANES = 2, 16, 16
NW = NC * NS
N = B * L_SEQ
PER_W = N // NW
CH = 128
NCHUNK = PER_W // CH
NG = CH // LANES
EBP = EMB // 2
CVP = 264
PSTR = CH + 1


def _body(wid, cids2, glove, ctab, out,
          idx0, idx1, cid0, cid1, rows0, rows1, comb0, comb1, pooled_v, ct_v,
          s_idx0, s_idx1, s_cid0, s_cid1, s_g0, s_g1, s_r0, s_r1, s_o0, s_o1):
    idx_v = (idx0, idx1)
    cid_v = (cid0, cid1)
    rows_v = (rows0, rows1)
    comb_v = (comb0, comb1)
    s_idx = (s_idx0, s_idx1)
    s_cid = (s_cid0, s_cid1)
    s_g = (s_g0, s_g1)
    s_r = (s_r0, s_r1)
    s_o = (s_o0, s_o1)

    w = lax.axis_index("s") * NC + lax.axis_index("c")
    pltpu.sync_copy(ctab, ct_v)
    ebase = [lax.iota(jnp.int32, LANES) * PSTR + j * LANES * PSTR
             for j in range(EMB // LANES)]

    def fire_in(kc, p):
        p0 = pl.multiple_of(w * PER_W + kc * CH, CH)
        pltpu.async_copy(wid.at[pl.ds(p0, CH)], idx_v[p], s_idx[p])
        l = lax.shift_right_logical(p0, 12)
        b0 = pl.multiple_of(jnp.bitwise_and(p0, B - 1), CH)
        pltpu.async_copy(cids2.at[pl.ds(l * C, C), pl.ds(b0, CH)],
                         cid_v[p], s_cid[p])

    def wait_in(p):
        pltpu.make_async_copy(wid.at[pl.ds(0, CH)], idx_v[p], s_idx[p]).wait()
        pltpu.make_async_copy(cids2.at[pl.ds(0, C), pl.ds(0, CH)],
                              cid_v[p], s_cid[p]).wait()

    def fire_glove(p):
        pltpu.async_copy(glove.at[idx_v[p]], rows_v[p], s_g[p])

    def wait_glove(p):
        pltpu.make_async_copy(glove.at[idx_v[p]], rows_v[p], s_g[p]).wait()

    def fire_rows_out(kc, p):
        p0 = w * PER_W + kc * CH
        pltpu.async_copy(rows_v[p], out.at[pl.ds(p0, CH)], s_r[p])

    def wait_rows_out(p):
        pltpu.make_async_copy(rows_v[p], out.at[pl.ds(0, CH)], s_r[p]).wait()

    def fire_out(kc, p):
        p0 = w * PER_W + kc * CH
        pltpu.async_copy(comb_v[p], out.at[pl.ds(p0, CH), pl.ds(EMB, EMB)],
                         s_o[p])

    def wait_out(p):
        pltpu.make_async_copy(comb_v[p], out.at[pl.ds(0, CH), pl.ds(EMB, EMB)],
                              s_o[p]).wait()

    def compute(p):
        cv = cid_v[p]
        ov = comb_v[p]

        def group_body(g, gc):
            gw = g * LANES
            for ebb in range(EBP // 8):
                accs = [None] * 8
                for c in range(C):
                    ids = cv[c, pl.ds(gw, LANES)]
                    for q in range(8):
                        eb = ebb * 8 + q
                        tab = ct_v.at[pl.ds(eb * CVP, CVP)]
                        v = plsc.bitcast(plsc.load_gather(tab, [ids]),
                                         jnp.bfloat16)
                        accs[q] = v if c == 0 else jnp.maximum(accs[q], v)
                for q in range(8):
                    eb = ebb * 8 + q
                    ai = plsc.bitcast(accs[q], jnp.int32)
                    lo = plsc.bitcast(lax.shift_left(ai, 16), jnp.float32)
                    hi = plsc.bitcast(jnp.bitwise_and(ai, jnp.int32(-65536)),
                                      jnp.float32)
                    pooled_v[pl.ds((2 * eb) * PSTR + gw, LANES)] = lo
                    pooled_v[pl.ds((2 * eb + 1) * PSTR + gw, LANES)] = hi
            for i in range(LANES):
                word = gw + i
                for j in range(EMB // LANES):
                    v = plsc.load_gather(pooled_v, [ebase[j] + word])
                    ov[word, pl.ds(j * LANES, LANES)] = v
            return gc

        lax.fori_loop(0, NG, group_body, 0)

    fire_in(0, 0)
    wait_in(0)
    fire_glove(0)
    fire_in(1, 1)

    def super_body(kk, carry):
        for ph in range(2):
            k = kk * 2 + ph
            p, p1 = ph, 1 - ph
            if ph == 0:
                wait_in(p1)

                @pl.when(kk >= 1)
                def _():
                    wait_out(p1)

                fire_glove(p1)
            else:
                @pl.when(kk < (NCHUNK // 2) - 1)
                def _():
                    wait_in(p1)
                    wait_out(p1)
                    fire_glove(p1)

            wait_glove(p)
            fire_rows_out(k, p)
            compute(p)
            wait_rows_out(p)
            fire_out(k, p)

            @pl.when(kk < (NCHUNK // 2) - 1)
            def _():
                fire_in(k + 2, p)
        return carry

    lax.fori_loop(0, NCHUNK // 2, super_body, 0)
    wait_out(0)
    wait_out(1)


def kernel(word_ids, char_ids, glove_table, char_table):
    wid = word_ids.T.reshape(N)
    glove128 = jnp.pad(glove_table, ((0, 0), (0, EMB)))
    cids2 = char_ids.transpose(1, 2, 0).reshape(L_SEQ * C, B)
    ct_bf = char_table.astype(jnp.bfloat16).reshape(CHAR_VOCAB, EBP, 2)
    ct_pk = lax.bitcast_convert_type(ct_bf, jnp.int32)
    ct_pk = jnp.pad(ct_pk.T, ((0, 0), (0, CVP - CHAR_VOCAB)))
    ct_pk = ct_pk.reshape(EBP * CVP)

    mesh = plsc.VectorSubcoreMesh(core_axis_name="c", subcore_axis_name="s")
    out = pl.kernel(
        _body,
        out_type=jax.ShapeDtypeStruct((N, 2 * EMB), jnp.float32),
        mesh=mesh,
        compiler_params=pltpu.CompilerParams(
            needs_layout_passes=False, use_tc_tiling_on_sc=False),
        scratch_types=[
            pltpu.VMEM((CH,), jnp.int32),
            pltpu.VMEM((CH,), jnp.int32),
            pltpu.VMEM((C, CH), jnp.int32),
            pltpu.VMEM((C, CH), jnp.int32),
            pltpu.VMEM((CH, 2 * EMB), jnp.float32),
            pltpu.VMEM((CH, 2 * EMB), jnp.float32),
            pltpu.VMEM((CH, EMB), jnp.float32),
            pltpu.VMEM((CH, EMB), jnp.float32),
            pltpu.VMEM((EMB * PSTR,), jnp.float32),
            pltpu.VMEM((EBP * CVP,), jnp.int32),
            pltpu.SemaphoreType.DMA,
            pltpu.SemaphoreType.DMA,
            pltpu.SemaphoreType.DMA,
            pltpu.SemaphoreType.DMA,
            pltpu.SemaphoreType.DMA,
            pltpu.SemaphoreType.DMA,
            pltpu.SemaphoreType.DMA,
            pltpu.SemaphoreType.DMA,
            pltpu.SemaphoreType.DMA,
            pltpu.SemaphoreType.DMA,
        ],
    )(wid, cids2, glove128, ct_pk)
    return out.reshape(L_SEQ, B, 2 * EMB).transpose(1, 0, 2)


# speedup vs baseline: 1.1517x; 1.0839x over previous
import jax
import jax.numpy as jnp
from jax import lax
from jax.experimental import pallas as pl
from jax.experimental.pallas import tpu as pltpu
from jax.experimental.pallas import tpu_sc as plsc

WORD_VOCAB = 1000000
CHAR_VOCAB = 262
EMB = 64
B = 4096
L_SEQ = 50
C = 16

NC, NS, LANES = 2, 16, 16
NW = NC * NS
N = B * L_SEQ
PER_W = N // NW
CH = 128
NCHUNK = PER_W // CH
NG = CH // LANES
EBP = EMB // 2
CVP = 264
PSTR = CH + 1


def _body(wid, cids2, glove, ctab, out,
          idx0, idx1, cid0, cid1, rows0, rows1, comb0, comb1, pooled_v, ct_v,
          s_idx0, s_idx1, s_cid0, s_cid1, s_g0, s_g1, s_r0, s_r1, s_o0, s_o1):
    idx_v = (idx0, idx1)
    cid_v = (cid0, cid1)
    rows_v = (rows0, rows1)
    comb_v = (comb0, comb1)
    s_idx = (s_idx0, s_idx1)
    s_cid = (s_cid0, s_cid1)
    s_g = (s_g0, s_g1)
    s_r = (s_r0, s_r1)
    s_o = (s_o0, s_o1)

    w = lax.axis_index("s") * NC + lax.axis_index("c")
    pltpu.sync_copy(ctab, ct_v)
    ebase = [lax.iota(jnp.int32, LANES) * PSTR + j * LANES * PSTR
             for j in range(EMB // LANES)]

    def fire_in(kc, p):
        p0 = pl.multiple_of(w * PER_W + kc * CH, CH)
        pltpu.async_copy(wid.at[pl.ds(p0, CH)], idx_v[p], s_idx[p])
        l = lax.shift_right_logical(p0, 12)
        b0 = pl.multiple_of(jnp.bitwise_and(p0, B - 1), CH)
        pltpu.async_copy(cids2.at[pl.ds(l * C, C), pl.ds(b0, CH)],
                         cid_v[p], s_cid[p])

    def wait_in(p):
        pltpu.make_async_copy(wid.at[pl.ds(0, CH)], idx_v[p], s_idx[p]).wait()
        pltpu.make_async_copy(cids2.at[pl.ds(0, C), pl.ds(0, CH)],
                              cid_v[p], s_cid[p]).wait()

    def fire_glove(p):
        pltpu.async_copy(glove.at[idx_v[p]], rows_v[p], s_g[p])

    def wait_glove(p):
        pltpu.make_async_copy(glove.at[idx_v[p]], rows_v[p], s_g[p]).wait()

    def fire_rows_out(kc, p):
        p0 = w * PER_W + kc * CH
        pltpu.async_copy(rows_v[p], out.at[pl.ds(p0, CH)], s_r[p])

    def wait_rows_out(p):
        pltpu.make_async_copy(rows_v[p], out.at[pl.ds(0, CH)], s_r[p]).wait()

    def fire_out(kc, p):
        p0 = w * PER_W + kc * CH
        pltpu.async_copy(comb_v[p], out.at[pl.ds(p0, CH), pl.ds(EMB, EMB)],
                         s_o[p])

    def wait_out(p):
        pltpu.make_async_copy(comb_v[p], out.at[pl.ds(0, CH), pl.ds(EMB, EMB)],
                              s_o[p]).wait()

    def compute(p):
        cv = cid_v[p]
        ov = comb_v[p]

        def group_body(g, gc):
            gw = g * LANES
            for ebb in range(EBP // 4):
                accs = [None] * 4
                for c in range(C):
                    ids = cv[c, pl.ds(gw, LANES)]
                    for q in range(4):
                        eb = ebb * 4 + q
                        tab = ct_v.at[pl.ds(eb * CVP, CVP)]
                        v = plsc.bitcast(plsc.load_gather(tab, [ids]),
                                         jnp.bfloat16)
                        accs[q] = v if c == 0 else jnp.maximum(accs[q], v)
                for q in range(4):
                    eb = ebb * 4 + q
                    ai = plsc.bitcast(accs[q], jnp.int32)
                    lo = plsc.bitcast(lax.shift_left(ai, 16), jnp.float32)
                    hi = plsc.bitcast(jnp.bitwise_and(ai, jnp.int32(-65536)),
                                      jnp.float32)
                    pooled_v[pl.ds((2 * eb) * PSTR + gw, LANES)] = lo
                    pooled_v[pl.ds((2 * eb + 1) * PSTR + gw, LANES)] = hi
            for i in range(LANES):
                word = gw + i
                for j in range(EMB // LANES):
                    v = plsc.load_gather(pooled_v, [ebase[j] + word])
                    ov[word, pl.ds(j * LANES, LANES)] = v
            return gc

        lax.fori_loop(0, NG, group_body, 0)

    fire_in(0, 0)
    wait_in(0)
    fire_glove(0)
    fire_in(1, 1)

    def super_body(kk, carry):
        for ph in range(2):
            k = kk * 2 + ph
            p, p1 = ph, 1 - ph
            if ph == 0:
                wait_in(p1)

                @pl.when(kk >= 1)
                def _():
                    wait_out(p1)

                fire_glove(p1)
            else:
                @pl.when(kk < (NCHUNK // 2) - 1)
                def _():
                    wait_in(p1)
                    wait_out(p1)
                    fire_glove(p1)

            wait_glove(p)
            fire_rows_out(k, p)
            compute(p)
            wait_rows_out(p)
            fire_out(k, p)

            @pl.when(kk < (NCHUNK // 2) - 1)
            def _():
                fire_in(k + 2, p)
        return carry

    lax.fori_loop(0, NCHUNK // 2, super_body, 0)
    wait_out(0)
    wait_out(1)


def kernel(word_ids, char_ids, glove_table, char_table):
    wid = word_ids.T.reshape(N)
    glove128 = jnp.pad(glove_table, ((0, 0), (0, EMB)))
    cids2 = char_ids.transpose(1, 2, 0).reshape(L_SEQ * C, B)
    ct_bf = char_table.astype(jnp.bfloat16).reshape(CHAR_VOCAB, EBP, 2)
    ct_pk = lax.bitcast_convert_type(ct_bf, jnp.int32)
    ct_pk = jnp.pad(ct_pk.T, ((0, 0), (0, CVP - CHAR_VOCAB)))
    ct_pk = ct_pk.reshape(EBP * CVP)

    mesh = plsc.VectorSubcoreMesh(core_axis_name="c", subcore_axis_name="s")
    out = pl.kernel(
        _body,
        out_type=jax.ShapeDtypeStruct((N, 2 * EMB), jnp.float32),
        mesh=mesh,
        compiler_params=pltpu.CompilerParams(
            needs_layout_passes=False, use_tc_tiling_on_sc=False),
        scratch_types=[
            pltpu.VMEM((CH,), jnp.int32),
            pltpu.VMEM((CH,), jnp.int32),
            pltpu.VMEM((C, CH), jnp.int32),
            pltpu.VMEM((C, CH), jnp.int32),
            pltpu.VMEM((CH, 2 * EMB), jnp.float32),
            pltpu.VMEM((CH, 2 * EMB), jnp.float32),
            pltpu.VMEM((CH, EMB), jnp.float32),
            pltpu.VMEM((CH, EMB), jnp.float32),
            pltpu.VMEM((EMB * PSTR,), jnp.float32),
            pltpu.VMEM((EBP * CVP,), jnp.int32),
            pltpu.SemaphoreType.DMA,
            pltpu.SemaphoreType.DMA,
            pltpu.SemaphoreType.DMA,
            pltpu.SemaphoreType.DMA,
            pltpu.SemaphoreType.DMA,
            pltpu.SemaphoreType.DMA,
            pltpu.SemaphoreType.DMA,
            pltpu.SemaphoreType.DMA,
            pltpu.SemaphoreType.DMA,
            pltpu.SemaphoreType.DMA,
        ],
    )(wid, cids2, glove128, ct_pk)
    return out.reshape(L_SEQ, B, 2 * EMB).transpose(1, 0, 2)
